# baseline (speedup 1.0000x reference)
import functools
import math

import jax
import jax.numpy as jnp
from jax import lax
from jax.experimental import pallas as pl
from jax.experimental.pallas import tpu as pltpu
from jax.experimental.pallas import tpu_sc as plsc

NUM_FIELDS = 26
FIELD_DIM = 100000
DIM = 16
EPS = 1e-5

NC = 2
NS = 16
NW = NC * NS
LANES = 16

CHUNK = 128
TBLK = 16384
SPLITS = [
    (0, 9, 0, 55),
    (9, 9, 54, 56),
    (18, 8, 109, 50),
]


def _transpose_kernel(src_ref, lin_ref, dst_ref, lin_out_ref):
    blk = src_ref.shape[1]
    for k in range(blk // 1024):
        x8 = jnp.concatenate(
            [src_ref[:, k * 1024 + j * 128: k * 1024 + (j + 1) * 128]
             for j in range(8)], axis=0)
        dst_ref[k * 128:(k + 1) * 128, :] = x8.T
    for p in range(blk // 1024):
        piece = jnp.concatenate(
            [lin_ref[:, p * 1024 + s * 128: p * 1024 + (s + 1) * 128]
             for s in range(8)], axis=0)
        lin_out_ref[p * 8:(p + 1) * 8, :] = piece


def _to_row_major(emb_t, lin_t, lo, nblk):
    br = TBLK * DIM // 128
    nrow_t2 = nblk * br
    nrow = nblk * (TBLK // 128)
    flat, lin_flat = pl.pallas_call(
        _transpose_kernel,
        grid=(nblk,),
        in_specs=[pl.BlockSpec((DIM, TBLK), lambda i: (0, i + lo)),
                  pl.BlockSpec((1, TBLK), lambda i: (0, i + lo))],
        out_specs=[pl.BlockSpec((br, 128), lambda i: (i, 0)),
                   pl.BlockSpec((TBLK // 128, 128), lambda i: (i, 0))],
        out_shape=[
            jax.ShapeDtypeStruct((nrow_t2, 128), jnp.float32),
            jax.ShapeDtypeStruct((nrow, 128), jnp.float32),
        ],
    )(emb_t, lin_t)
    return (flat.reshape(nrow_t2 * 128 // DIM, DIM),
            lin_flat.reshape(nrow * 128))


def _sc_gather_part(xi_flat, xip_flat, emb_table, lin_flat, batch, nf):
    per_w = batch // NW
    n_chunks = per_w // CHUNK
    rows = CHUNK * nf
    idx_rows = rows // 128
    mesh = plsc.VectorSubcoreMesh(core_axis_name="c", subcore_axis_name="s")

    @functools.partial(
        pl.kernel,
        out_type=[
            jax.ShapeDtypeStruct((batch, DIM), jnp.float32),
            jax.ShapeDtypeStruct((batch, DIM), jnp.float32),
            jax.ShapeDtypeStruct((batch * nf,), jnp.float32),
        ],
        mesh=mesh,
        compiler_params=pltpu.CompilerParams(use_tc_tiling_on_sc=False),
        scratch_types=[
            pltpu.VMEM((rows,), jnp.int32),
            pltpu.VMEM((rows,), jnp.int32),
            pltpu.VMEM((rows, DIM), jnp.float32),
            pltpu.VMEM((rows,), jnp.float32),
            pltpu.VMEM((CHUNK, DIM), jnp.float32),
            pltpu.VMEM((CHUNK, DIM), jnp.float32),
            pltpu.SemaphoreType.DMA,
            pltpu.SemaphoreType.DMA,
        ],
    )
    def sc_kernel(xi_hbm, xip_hbm, emb_hbm, lin_hbm, s_hbm, sq_hbm,
                  linval_hbm, idx_v, idxp_v, rows_v, linv_v, s_v, sq_v,
                  sem_e, sem_l):
        wid = lax.axis_index("s") * NC + lax.axis_index("c")

        def chunk_body(c, _):
            base_e = wid * per_w + c * CHUNK
            i0 = base_e * nf

            pltpu.sync_copy(xi_hbm.at[pl.ds(i0, rows)], idx_v)
            pltpu.sync_copy(xip_hbm.at[pl.ds(i0, rows)], idxp_v)

            copies = []
            for j in range(idx_rows):
                copies.append(pltpu.async_copy(
                    emb_hbm.at[idxp_v.at[pl.ds(j * 128, 128)]],
                    rows_v.at[pl.ds(j * 128, 128)], sem_e))
            for j in range(idx_rows):
                copies.append(pltpu.async_copy(
                    lin_hbm.at[idx_v.at[pl.ds(j * 128, 128)]],
                    linv_v.at[pl.ds(j * 128, 128)], sem_l))
            for cp in copies:
                cp.wait()

            zero = jnp.zeros((LANES,), jnp.float32)

            def elem_body(e, _):
                s = zero
                sq = zero
                base = e * nf
                for f in range(nf):
                    v = rows_v[base + f]
                    s = s + v
                    sq = sq + v * v
                s_v[e] = s
                sq_v[e] = sq
                return 0

            lax.fori_loop(0, CHUNK, elem_body, 0, unroll=False)

            pltpu.sync_copy(s_v, s_hbm.at[pl.ds(base_e, CHUNK)])
            pltpu.sync_copy(sq_v, sq_hbm.at[pl.ds(base_e, CHUNK)])
            pltpu.sync_copy(linv_v, linval_hbm.at[pl.ds(i0, rows)])
            return 0

        lax.fori_loop(0, n_chunks, chunk_body, 0, unroll=False)

    return sc_kernel(xi_flat, xip_flat, emb_table, lin_flat)


def _mlp_kernel(s1_ref, sq1_ref, lv1_ref, s2_ref, sq2_ref, lv2_ref,
                s3_ref, sq3_ref, lv3_ref,
                bn0g_ref, bn0b_ref, w1_ref, b1_ref,
                bn1g_ref, bn1b_ref, w2_ref, b2_ref, bn2g_ref, bn2b_ref,
                wo_ref, const_ref, out_ref):
    inv = jnp.float32(1.0 / math.sqrt(1.0 + EPS))
    s = s1_ref[...] + s2_ref[...] + s3_ref[...]
    sq = sq1_ref[...] + sq2_ref[...] + sq3_ref[...]
    cross = 0.5 * (s * s - sq)
    lin = (jnp.sum(lv1_ref[...], axis=1, keepdims=True)
           + jnp.sum(lv2_ref[...], axis=1, keepdims=True)
           + jnp.sum(lv3_ref[...], axis=1, keepdims=True))
    xb = cross * (bn0g_ref[...] * inv) + bn0b_ref[...]
    h = jnp.dot(xb, w1_ref[...], preferred_element_type=jnp.float32)
    h = (h + b1_ref[...]) * (bn1g_ref[...] * inv) + bn1b_ref[...]
    h = jnp.maximum(h, 0.0)
    h = jnp.dot(h, w2_ref[...], preferred_element_type=jnp.float32)
    h = (h + b2_ref[...]) * (bn2g_ref[...] * inv) + bn2b_ref[...]
    h = jnp.maximum(h, 0.0)
    o = jnp.dot(h, wo_ref[...], preferred_element_type=jnp.float32)
    out_ref[...] = o + lin + const_ref[...]


def _perm(r):
    return (r // 1024) * 1024 + (r % 128) * 8 + (r // 128) % 8


def kernel(x, emb_table, lin_table, lin_bias, bn0_g, bn0_b, W1, b1,
           bn1_g, bn1_b, W2, b2, bn2_g, bn2_b, Wo, bo):
    batch = x.shape[0]
    offsets = (jnp.arange(NUM_FIELDS) * FIELD_DIM).astype(jnp.int32)
    xi2d = x.astype(jnp.int32) + offsets[None, :]

    parts = []
    for f0, nf, lo, nblk in SPLITS:
        xi_p = xi2d[:, f0:f0 + nf].reshape(-1) - lo * TBLK
        emb_p, lin_p = _to_row_major(emb_table.T, lin_table.T, lo, nblk)
        parts.append(_sc_gather_part(xi_p, _perm(xi_p), emb_p, lin_p,
                                     batch, nf))
    (s1, sq1, lv1), (s2, sq2, lv2), (s3, sq3, lv3) = parts

    bk = 2048
    grid = (batch // bk,)
    row = lambda a: a.reshape(1, -1)
    full = lambda shape: pl.BlockSpec(shape, lambda i: (0, 0))
    const = (lin_bias + bo).reshape(1, 1)

    out = pl.pallas_call(
        _mlp_kernel,
        grid=grid,
        in_specs=[
            pl.BlockSpec((bk, DIM), lambda i: (i, 0)),
            pl.BlockSpec((bk, DIM), lambda i: (i, 0)),
            pl.BlockSpec((bk, SPLITS[0][1]), lambda i: (i, 0)),
            pl.BlockSpec((bk, DIM), lambda i: (i, 0)),
            pl.BlockSpec((bk, DIM), lambda i: (i, 0)),
            pl.BlockSpec((bk, SPLITS[1][1]), lambda i: (i, 0)),
            pl.BlockSpec((bk, DIM), lambda i: (i, 0)),
            pl.BlockSpec((bk, DIM), lambda i: (i, 0)),
            pl.BlockSpec((bk, SPLITS[2][1]), lambda i: (i, 0)),
            full((1, DIM)), full((1, DIM)),
            full((DIM, 64)), full((1, 64)), full((1, 64)), full((1, 64)),
            full((64, 32)), full((1, 32)), full((1, 32)), full((1, 32)),
            full((32, 1)), full((1, 1)),
        ],
        out_specs=pl.BlockSpec((bk, 1), lambda i: (i, 0)),
        out_shape=jax.ShapeDtypeStruct((batch, 1), jnp.float32),
    )(s1, sq1, lv1.reshape(batch, SPLITS[0][1]),
      s2, sq2, lv2.reshape(batch, SPLITS[1][1]),
      s3, sq3, lv3.reshape(batch, SPLITS[2][1]),
      row(bn0_g), row(bn0_b), W1, row(b1),
      row(bn1_g), row(bn1_b), W2, row(b2), row(bn2_g), row(bn2_b), Wo, const)

    return out.reshape(batch)


# speedup vs baseline: 1.0951x; 1.0951x over previous
import functools
import math

import jax
import jax.numpy as jnp
from jax import lax
from jax.experimental import pallas as pl
from jax.experimental.pallas import tpu as pltpu
from jax.experimental.pallas import tpu_sc as plsc

NUM_FIELDS = 26
FIELD_DIM = 100000
DIM = 16
EPS = 1e-5

NC = 2
NS = 16
NW = NC * NS
LANES = 16

TBLK = 16384
SPLITS = [
    (0, NUM_FIELDS, 0, 159),
]


def _transpose_kernel(src_ref, lin_ref, dst_ref, lin_out_ref):
    blk = src_ref.shape[1]
    for k in range(blk // 1024):
        x8 = jnp.concatenate(
            [src_ref[:, k * 1024 + j * 128: k * 1024 + (j + 1) * 128]
             for j in range(8)], axis=0)
        dst_ref[k * 128:(k + 1) * 128, :] = x8.T
    for p in range(blk // 1024):
        piece = jnp.concatenate(
            [lin_ref[:, p * 1024 + s * 128: p * 1024 + (s + 1) * 128]
             for s in range(8)], axis=0)
        lin_out_ref[p * 8:(p + 1) * 8, :] = piece


def _to_row_major(emb_t, lin_t, lo, nblk):
    br = TBLK * DIM // 128
    nrow_t2 = nblk * br
    nrow = nblk * (TBLK // 128)
    flat, lin_flat = pl.pallas_call(
        _transpose_kernel,
        grid=(nblk,),
        in_specs=[pl.BlockSpec((DIM, TBLK), lambda i: (0, i + lo)),
                  pl.BlockSpec((1, TBLK), lambda i: (0, i + lo))],
        out_specs=[pl.BlockSpec((br, 128), lambda i: (i, 0)),
                   pl.BlockSpec((TBLK // 128, 128), lambda i: (i, 0))],
        out_shape=[
            jax.ShapeDtypeStruct((nrow_t2, 128), jnp.float32),
            jax.ShapeDtypeStruct((nrow, 128), jnp.float32),
        ],
    )(emb_t, lin_t)
    return (flat.reshape(nrow_t2 * 128 // DIM, DIM),
            lin_flat.reshape(nrow * 128))


def _sc_gather_part(xi_flat, xip_flat, emb_table, lin_flat, batch, nf):
    per_w = batch // NW
    chunk = 64 if nf >= 16 else 128
    n_chunks = per_w // chunk
    rows = chunk * nf
    idx_rows = rows // 128
    mesh = plsc.VectorSubcoreMesh(core_axis_name="c", subcore_axis_name="s")

    @functools.partial(
        pl.kernel,
        out_type=[
            jax.ShapeDtypeStruct((batch, DIM), jnp.float32),
            jax.ShapeDtypeStruct((batch, DIM), jnp.float32),
            jax.ShapeDtypeStruct((batch * nf,), jnp.float32),
        ],
        mesh=mesh,
        compiler_params=pltpu.CompilerParams(use_tc_tiling_on_sc=False),
        scratch_types=[
            pltpu.VMEM((rows,), jnp.int32),
            pltpu.VMEM((rows,), jnp.int32),
            pltpu.VMEM((rows, DIM), jnp.float32),
            pltpu.VMEM((rows,), jnp.float32),
            pltpu.VMEM((chunk, DIM), jnp.float32),
            pltpu.VMEM((chunk, DIM), jnp.float32),
            pltpu.SemaphoreType.DMA,
            pltpu.SemaphoreType.DMA,
        ],
    )
    def sc_kernel(xi_hbm, xip_hbm, emb_hbm, lin_hbm, s_hbm, sq_hbm,
                  linval_hbm, idx_v, idxp_v, rows_v, linv_v, s_v, sq_v,
                  sem_e, sem_l):
        wid = lax.axis_index("s") * NC + lax.axis_index("c")

        def chunk_body(c, _):
            base_e = wid * per_w + c * chunk
            i0 = base_e * nf

            pltpu.sync_copy(xi_hbm.at[pl.ds(i0, rows)], idx_v)
            pltpu.sync_copy(xip_hbm.at[pl.ds(i0, rows)], idxp_v)

            copies = []
            for j in range(idx_rows):
                copies.append(pltpu.async_copy(
                    emb_hbm.at[idxp_v.at[pl.ds(j * 128, 128)]],
                    rows_v.at[pl.ds(j * 128, 128)], sem_e))
            for j in range(idx_rows):
                copies.append(pltpu.async_copy(
                    lin_hbm.at[idx_v.at[pl.ds(j * 128, 128)]],
                    linv_v.at[pl.ds(j * 128, 128)], sem_l))
            for cp in copies:
                cp.wait()

            zero = jnp.zeros((LANES,), jnp.float32)

            def elem_body(e, _):
                s = zero
                sq = zero
                base = e * nf
                for f in range(nf):
                    v = rows_v[base + f]
                    s = s + v
                    sq = sq + v * v
                s_v[e] = s
                sq_v[e] = sq
                return 0

            lax.fori_loop(0, chunk, elem_body, 0, unroll=False)

            pltpu.sync_copy(s_v, s_hbm.at[pl.ds(base_e, chunk)])
            pltpu.sync_copy(sq_v, sq_hbm.at[pl.ds(base_e, chunk)])
            pltpu.sync_copy(linv_v, linval_hbm.at[pl.ds(i0, rows)])
            return 0

        lax.fori_loop(0, n_chunks, chunk_body, 0, unroll=False)

    return sc_kernel(xi_flat, xip_flat, emb_table, lin_flat)


def _mlp_kernel(nparts, *refs):
    (bn0g_ref, bn0b_ref, w1_ref, b1_ref, bn1g_ref, bn1b_ref, w2_ref,
     b2_ref, bn2g_ref, bn2b_ref, wo_ref, const_ref,
     out_ref) = refs[3 * nparts:]
    inv = jnp.float32(1.0 / math.sqrt(1.0 + EPS))
    s = sum(refs[3 * p][...] for p in range(nparts))
    sq = sum(refs[3 * p + 1][...] for p in range(nparts))
    cross = 0.5 * (s * s - sq)
    lin = sum(jnp.sum(refs[3 * p + 2][...], axis=1, keepdims=True)
              for p in range(nparts))
    xb = cross * (bn0g_ref[...] * inv) + bn0b_ref[...]
    h = jnp.dot(xb, w1_ref[...], preferred_element_type=jnp.float32)
    h = (h + b1_ref[...]) * (bn1g_ref[...] * inv) + bn1b_ref[...]
    h = jnp.maximum(h, 0.0)
    h = jnp.dot(h, w2_ref[...], preferred_element_type=jnp.float32)
    h = (h + b2_ref[...]) * (bn2g_ref[...] * inv) + bn2b_ref[...]
    h = jnp.maximum(h, 0.0)
    o = jnp.dot(h, wo_ref[...], preferred_element_type=jnp.float32)
    out_ref[...] = o + lin + const_ref[...]


def _perm(r):
    return (r // 1024) * 1024 + (r % 128) * 8 + (r // 128) % 8


def kernel(x, emb_table, lin_table, lin_bias, bn0_g, bn0_b, W1, b1,
           bn1_g, bn1_b, W2, b2, bn2_g, bn2_b, Wo, bo):
    batch = x.shape[0]
    offsets = (jnp.arange(NUM_FIELDS) * FIELD_DIM).astype(jnp.int32)
    xi2d = x.astype(jnp.int32) + offsets[None, :]

    parts = []
    for f0, nf, lo, nblk in SPLITS:
        xi_p = xi2d[:, f0:f0 + nf].reshape(-1) - lo * TBLK
        emb_p, lin_p = _to_row_major(emb_table.T, lin_table.T, lo, nblk)
        parts.append(_sc_gather_part(xi_p, _perm(xi_p), emb_p, lin_p,
                                     batch, nf))
    bk = 2048
    grid = (batch // bk,)
    row = lambda a: a.reshape(1, -1)
    blk2 = lambda w: pl.BlockSpec((bk, w), lambda i: (i, 0))
    full = lambda shape: pl.BlockSpec(shape, lambda i: (0, 0))
    const = (lin_bias + bo).reshape(1, 1)

    part_specs = []
    part_args = []
    for (f0, nf, lo, nblk), (s_p, sq_p, lv_p) in zip(SPLITS, parts):
        part_specs += [blk2(DIM), blk2(DIM), blk2(nf)]
        part_args += [s_p, sq_p, lv_p.reshape(batch, nf)]

    out = pl.pallas_call(
        functools.partial(_mlp_kernel, len(SPLITS)),
        grid=grid,
        in_specs=part_specs + [
            full((1, DIM)), full((1, DIM)),
            full((DIM, 64)), full((1, 64)), full((1, 64)), full((1, 64)),
            full((64, 32)), full((1, 32)), full((1, 32)), full((1, 32)),
            full((32, 1)), full((1, 1)),
        ],
        out_specs=pl.BlockSpec((bk, 1), lambda i: (i, 0)),
        out_shape=jax.ShapeDtypeStruct((batch, 1), jnp.float32),
    )(*part_args, row(bn0_g), row(bn0_b), W1, row(b1),
      row(bn1_g), row(bn1_b), W2, row(b2), row(bn2_g), row(bn2_b), Wo, const)

    return out.reshape(batch)


# speedup vs baseline: 1.2932x; 1.1809x over previous
import functools
import math

import jax
import jax.numpy as jnp
from jax import lax
from jax.experimental import pallas as pl
from jax.experimental.pallas import tpu as pltpu
from jax.experimental.pallas import tpu_sc as plsc

NUM_FIELDS = 26
FIELD_DIM = 100000
DIM = 16
EPS = 1e-5

NC = 2
NS = 16
NW = NC * NS
LANES = 16

TBLK = 32768
SPLITS = [
    (0, NUM_FIELDS, 0, 80),
]


def _transpose_kernel(src_ref, lin_ref, dst_ref, lin_out_ref):
    blk = src_ref.shape[1]
    for k in range(blk // 1024):
        x8 = jnp.concatenate(
            [src_ref[:, k * 1024 + j * 128: k * 1024 + (j + 1) * 128]
             for j in range(8)], axis=0)
        dst_ref[k * 128:(k + 1) * 128, :] = x8.T
    for p in range(blk // 1024):
        piece = jnp.concatenate(
            [lin_ref[:, p * 1024 + s * 128: p * 1024 + (s + 1) * 128]
             for s in range(8)], axis=0)
        lin_out_ref[p * 8:(p + 1) * 8, :] = piece


def _to_row_major(emb_t, lin_t, lo, nblk):
    br = TBLK * DIM // 128
    nrow_t2 = nblk * br
    nrow = nblk * (TBLK // 128)
    flat, lin_flat = pl.pallas_call(
        _transpose_kernel,
        grid=(nblk,),
        in_specs=[pl.BlockSpec((DIM, TBLK), lambda i: (0, i + lo)),
                  pl.BlockSpec((1, TBLK), lambda i: (0, i + lo))],
        out_specs=[pl.BlockSpec((br, 128), lambda i: (i, 0)),
                   pl.BlockSpec((TBLK // 128, 128), lambda i: (i, 0))],
        out_shape=[
            jax.ShapeDtypeStruct((nrow_t2, 128), jnp.float32),
            jax.ShapeDtypeStruct((nrow, 128), jnp.float32),
        ],
    )(emb_t, lin_t)
    return (flat.reshape(nrow_t2 * 128 // DIM, DIM),
            lin_flat.reshape(nrow * 128))


def _sc_gather_part(xi_flat, xip_flat, emb_table, lin_flat, batch, nf):
    per_w = batch // NW
    chunk = 64 if nf >= 16 else 128
    n_chunks = per_w // chunk
    rows = chunk * nf
    idx_rows = rows // 128
    mesh = plsc.VectorSubcoreMesh(core_axis_name="c", subcore_axis_name="s")

    @functools.partial(
        pl.kernel,
        out_type=[
            jax.ShapeDtypeStruct((batch, DIM), jnp.float32),
            jax.ShapeDtypeStruct((batch, DIM), jnp.float32),
            jax.ShapeDtypeStruct((batch * nf,), jnp.float32),
        ],
        mesh=mesh,
        compiler_params=pltpu.CompilerParams(use_tc_tiling_on_sc=False),
        scratch_types=[
            pltpu.VMEM((rows,), jnp.int32),
            pltpu.VMEM((rows,), jnp.int32),
            pltpu.VMEM((rows, DIM), jnp.float32),
            pltpu.VMEM((rows,), jnp.float32),
            pltpu.VMEM((chunk, DIM), jnp.float32),
            pltpu.VMEM((chunk, DIM), jnp.float32),
            pltpu.SemaphoreType.DMA,
            pltpu.SemaphoreType.DMA,
        ],
    )
    def sc_kernel(xi_hbm, xip_hbm, emb_hbm, lin_hbm, s_hbm, sq_hbm,
                  linval_hbm, idx_v, idxp_v, rows_v, linv_v, s_v, sq_v,
                  sem_e, sem_l):
        wid = lax.axis_index("s") * NC + lax.axis_index("c")

        def chunk_body(c, _):
            base_e = wid * per_w + c * chunk
            i0 = base_e * nf

            pltpu.sync_copy(xi_hbm.at[pl.ds(i0, rows)], idx_v)
            pltpu.sync_copy(xip_hbm.at[pl.ds(i0, rows)], idxp_v)

            copies = []
            for j in range(idx_rows):
                copies.append(pltpu.async_copy(
                    emb_hbm.at[idxp_v.at[pl.ds(j * 128, 128)]],
                    rows_v.at[pl.ds(j * 128, 128)], sem_e))
            for j in range(idx_rows):
                copies.append(pltpu.async_copy(
                    lin_hbm.at[idx_v.at[pl.ds(j * 128, 128)]],
                    linv_v.at[pl.ds(j * 128, 128)], sem_l))
            for cp in copies:
                cp.wait()

            zero = jnp.zeros((LANES,), jnp.float32)

            def elem_body(e, _):
                s = zero
                sq = zero
                base = e * nf
                for f in range(nf):
                    v = rows_v[base + f]
                    s = s + v
                    sq = sq + v * v
                s_v[e] = s
                sq_v[e] = sq
                return 0

            lax.fori_loop(0, chunk, elem_body, 0, unroll=False)

            pltpu.sync_copy(s_v, s_hbm.at[pl.ds(base_e, chunk)])
            pltpu.sync_copy(sq_v, sq_hbm.at[pl.ds(base_e, chunk)])
            pltpu.sync_copy(linv_v, linval_hbm.at[pl.ds(i0, rows)])
            return 0

        lax.fori_loop(0, n_chunks, chunk_body, 0, unroll=False)

    return sc_kernel(xi_flat, xip_flat, emb_table, lin_flat)


def _mlp_kernel(nparts, *refs):
    (bn0g_ref, bn0b_ref, w1_ref, b1_ref, bn1g_ref, bn1b_ref, w2_ref,
     b2_ref, bn2g_ref, bn2b_ref, wo_ref, const_ref,
     out_ref) = refs[3 * nparts:]
    inv = jnp.float32(1.0 / math.sqrt(1.0 + EPS))
    s = sum(refs[3 * p][...] for p in range(nparts))
    sq = sum(refs[3 * p + 1][...] for p in range(nparts))
    cross = 0.5 * (s * s - sq)
    lin = sum(jnp.sum(refs[3 * p + 2][...], axis=1, keepdims=True)
              for p in range(nparts))
    xb = cross * (bn0g_ref[...] * inv) + bn0b_ref[...]
    h = jnp.dot(xb, w1_ref[...], preferred_element_type=jnp.float32)
    h = (h + b1_ref[...]) * (bn1g_ref[...] * inv) + bn1b_ref[...]
    h = jnp.maximum(h, 0.0)
    h = jnp.dot(h, w2_ref[...], preferred_element_type=jnp.float32)
    h = (h + b2_ref[...]) * (bn2g_ref[...] * inv) + bn2b_ref[...]
    h = jnp.maximum(h, 0.0)
    o = jnp.dot(h, wo_ref[...], preferred_element_type=jnp.float32)
    out_ref[...] = o + lin + const_ref[...]


def _perm(r):
    return (r // 1024) * 1024 + (r % 128) * 8 + (r // 128) % 8


def kernel(x, emb_table, lin_table, lin_bias, bn0_g, bn0_b, W1, b1,
           bn1_g, bn1_b, W2, b2, bn2_g, bn2_b, Wo, bo):
    batch = x.shape[0]
    offsets = (jnp.arange(NUM_FIELDS) * FIELD_DIM).astype(jnp.int32)
    xi2d = x.astype(jnp.int32) + offsets[None, :]

    parts = []
    for f0, nf, lo, nblk in SPLITS:
        xi_p = xi2d[:, f0:f0 + nf].reshape(-1) - lo * TBLK
        emb_p, lin_p = _to_row_major(emb_table.T, lin_table.T, lo, nblk)
        parts.append(_sc_gather_part(xi_p, _perm(xi_p), emb_p, lin_p,
                                     batch, nf))
    bk = 2048
    grid = (batch // bk,)
    row = lambda a: a.reshape(1, -1)
    blk2 = lambda w: pl.BlockSpec((bk, w), lambda i: (i, 0))
    full = lambda shape: pl.BlockSpec(shape, lambda i: (0, 0))
    const = (lin_bias + bo).reshape(1, 1)

    part_specs = []
    part_args = []
    for (f0, nf, lo, nblk), (s_p, sq_p, lv_p) in zip(SPLITS, parts):
        part_specs += [blk2(DIM), blk2(DIM), blk2(nf)]
        part_args += [s_p, sq_p, lv_p.reshape(batch, nf)]

    out = pl.pallas_call(
        functools.partial(_mlp_kernel, len(SPLITS)),
        grid=grid,
        in_specs=part_specs + [
            full((1, DIM)), full((1, DIM)),
            full((DIM, 64)), full((1, 64)), full((1, 64)), full((1, 64)),
            full((64, 32)), full((1, 32)), full((1, 32)), full((1, 32)),
            full((32, 1)), full((1, 1)),
        ],
        out_specs=pl.BlockSpec((bk, 1), lambda i: (i, 0)),
        out_shape=jax.ShapeDtypeStruct((batch, 1), jnp.float32),
    )(*part_args, row(bn0_g), row(bn0_b), W1, row(b1),
      row(bn1_g), row(bn1_b), W2, row(b2), row(bn2_g), row(bn2_b), Wo, const)

    return out.reshape(batch)


# speedup vs baseline: 1.3773x; 1.0650x over previous
import functools
import math

import jax
import jax.numpy as jnp
from jax import lax
from jax.experimental import pallas as pl
from jax.experimental.pallas import tpu as pltpu
from jax.experimental.pallas import tpu_sc as plsc

NUM_FIELDS = 26
FIELD_DIM = 100000
DIM = 16
EPS = 1e-5

NC = 2
NS = 16
NW = NC * NS
LANES = 16

TBLK = 65536
SPLITS = [
    (0, NUM_FIELDS, 0, 40),
]


def _transpose_kernel(src_ref, lin_ref, dst_ref, lin_out_ref):
    blk = src_ref.shape[1]
    for k in range(blk // 1024):
        x8 = jnp.concatenate(
            [src_ref[:, k * 1024 + j * 128: k * 1024 + (j + 1) * 128]
             for j in range(8)], axis=0)
        dst_ref[k * 128:(k + 1) * 128, :] = x8.T
    for p in range(blk // 1024):
        piece = jnp.concatenate(
            [lin_ref[:, p * 1024 + s * 128: p * 1024 + (s + 1) * 128]
             for s in range(8)], axis=0)
        lin_out_ref[p * 8:(p + 1) * 8, :] = piece


def _to_row_major(emb_t, lin_t, lo, nblk):
    br = TBLK * DIM // 128
    nrow_t2 = nblk * br
    nrow = nblk * (TBLK // 128)
    flat, lin_flat = pl.pallas_call(
        _transpose_kernel,
        grid=(nblk,),
        in_specs=[pl.BlockSpec((DIM, TBLK), lambda i: (0, i + lo)),
                  pl.BlockSpec((1, TBLK), lambda i: (0, i + lo))],
        out_specs=[pl.BlockSpec((br, 128), lambda i: (i, 0)),
                   pl.BlockSpec((TBLK // 128, 128), lambda i: (i, 0))],
        out_shape=[
            jax.ShapeDtypeStruct((nrow_t2, 128), jnp.float32),
            jax.ShapeDtypeStruct((nrow, 128), jnp.float32),
        ],
    )(emb_t, lin_t)
    return (flat.reshape(nrow_t2 * 128 // DIM, DIM),
            lin_flat.reshape(nrow * 128))


def _sc_gather_part(xi_flat, xip_flat, emb_table, lin_flat, batch, nf):
    per_w = batch // NW
    chunk = 64 if nf >= 16 else 128
    n_chunks = per_w // chunk
    rows = chunk * nf
    idx_rows = rows // 128
    mesh = plsc.VectorSubcoreMesh(core_axis_name="c", subcore_axis_name="s")

    @functools.partial(
        pl.kernel,
        out_type=[
            jax.ShapeDtypeStruct((batch, DIM), jnp.float32),
            jax.ShapeDtypeStruct((batch, DIM), jnp.float32),
            jax.ShapeDtypeStruct((batch * nf,), jnp.float32),
        ],
        mesh=mesh,
        compiler_params=pltpu.CompilerParams(use_tc_tiling_on_sc=False),
        scratch_types=[
            pltpu.VMEM((rows,), jnp.int32),
            pltpu.VMEM((rows,), jnp.int32),
            pltpu.VMEM((rows, DIM), jnp.float32),
            pltpu.VMEM((rows,), jnp.float32),
            pltpu.VMEM((chunk, DIM), jnp.float32),
            pltpu.VMEM((chunk, DIM), jnp.float32),
            pltpu.SemaphoreType.DMA,
            pltpu.SemaphoreType.DMA,
        ],
    )
    def sc_kernel(xi_hbm, xip_hbm, emb_hbm, lin_hbm, s_hbm, sq_hbm,
                  linval_hbm, idx_v, idxp_v, rows_v, linv_v, s_v, sq_v,
                  sem_e, sem_l):
        wid = lax.axis_index("s") * NC + lax.axis_index("c")

        def chunk_body(c, _):
            base_e = wid * per_w + c * chunk
            i0 = base_e * nf

            pltpu.sync_copy(xi_hbm.at[pl.ds(i0, rows)], idx_v)
            pltpu.sync_copy(xip_hbm.at[pl.ds(i0, rows)], idxp_v)

            copies = []
            for j in range(idx_rows):
                copies.append(pltpu.async_copy(
                    emb_hbm.at[idxp_v.at[pl.ds(j * 128, 128)]],
                    rows_v.at[pl.ds(j * 128, 128)], sem_e))
            for j in range(idx_rows):
                copies.append(pltpu.async_copy(
                    lin_hbm.at[idx_v.at[pl.ds(j * 128, 128)]],
                    linv_v.at[pl.ds(j * 128, 128)], sem_l))
            for cp in copies:
                cp.wait()

            zero = jnp.zeros((LANES,), jnp.float32)

            def elem_body(e, _):
                s = zero
                sq = zero
                base = e * nf
                for f in range(nf):
                    v = rows_v[base + f]
                    s = s + v
                    sq = sq + v * v
                s_v[e] = s
                sq_v[e] = sq
                return 0

            lax.fori_loop(0, chunk, elem_body, 0, unroll=False)

            pltpu.sync_copy(s_v, s_hbm.at[pl.ds(base_e, chunk)])
            pltpu.sync_copy(sq_v, sq_hbm.at[pl.ds(base_e, chunk)])
            pltpu.sync_copy(linv_v, linval_hbm.at[pl.ds(i0, rows)])
            return 0

        lax.fori_loop(0, n_chunks, chunk_body, 0, unroll=False)

    return sc_kernel(xi_flat, xip_flat, emb_table, lin_flat)


def _mlp_kernel(nparts, *refs):
    (bn0g_ref, bn0b_ref, w1_ref, b1_ref, bn1g_ref, bn1b_ref, w2_ref,
     b2_ref, bn2g_ref, bn2b_ref, wo_ref, const_ref,
     out_ref) = refs[3 * nparts:]
    inv = jnp.float32(1.0 / math.sqrt(1.0 + EPS))
    s = sum(refs[3 * p][...] for p in range(nparts))
    sq = sum(refs[3 * p + 1][...] for p in range(nparts))
    cross = 0.5 * (s * s - sq)
    lin = sum(jnp.sum(refs[3 * p + 2][...], axis=1, keepdims=True)
              for p in range(nparts))
    xb = cross * (bn0g_ref[...] * inv) + bn0b_ref[...]
    h = jnp.dot(xb, w1_ref[...], preferred_element_type=jnp.float32)
    h = (h + b1_ref[...]) * (bn1g_ref[...] * inv) + bn1b_ref[...]
    h = jnp.maximum(h, 0.0)
    h = jnp.dot(h, w2_ref[...], preferred_element_type=jnp.float32)
    h = (h + b2_ref[...]) * (bn2g_ref[...] * inv) + bn2b_ref[...]
    h = jnp.maximum(h, 0.0)
    o = jnp.dot(h, wo_ref[...], preferred_element_type=jnp.float32)
    out_ref[...] = o + lin + const_ref[...]


def _perm(r):
    return (r // 1024) * 1024 + (r % 128) * 8 + (r // 128) % 8


def kernel(x, emb_table, lin_table, lin_bias, bn0_g, bn0_b, W1, b1,
           bn1_g, bn1_b, W2, b2, bn2_g, bn2_b, Wo, bo):
    batch = x.shape[0]
    offsets = (jnp.arange(NUM_FIELDS) * FIELD_DIM).astype(jnp.int32)
    xi2d = x.astype(jnp.int32) + offsets[None, :]

    parts = []
    for f0, nf, lo, nblk in SPLITS:
        xi_p = xi2d[:, f0:f0 + nf].reshape(-1) - lo * TBLK
        emb_p, lin_p = _to_row_major(emb_table.T, lin_table.T, lo, nblk)
        parts.append(_sc_gather_part(xi_p, _perm(xi_p), emb_p, lin_p,
                                     batch, nf))
    bk = 2048
    grid = (batch // bk,)
    row = lambda a: a.reshape(1, -1)
    blk2 = lambda w: pl.BlockSpec((bk, w), lambda i: (i, 0))
    full = lambda shape: pl.BlockSpec(shape, lambda i: (0, 0))
    const = (lin_bias + bo).reshape(1, 1)

    part_specs = []
    part_args = []
    for (f0, nf, lo, nblk), (s_p, sq_p, lv_p) in zip(SPLITS, parts):
        part_specs += [blk2(DIM), blk2(DIM), blk2(nf)]
        part_args += [s_p, sq_p, lv_p.reshape(batch, nf)]

    out = pl.pallas_call(
        functools.partial(_mlp_kernel, len(SPLITS)),
        grid=grid,
        in_specs=part_specs + [
            full((1, DIM)), full((1, DIM)),
            full((DIM, 64)), full((1, 64)), full((1, 64)), full((1, 64)),
            full((64, 32)), full((1, 32)), full((1, 32)), full((1, 32)),
            full((32, 1)), full((1, 1)),
        ],
        out_specs=pl.BlockSpec((bk, 1), lambda i: (i, 0)),
        out_shape=jax.ShapeDtypeStruct((batch, 1), jnp.float32),
    )(*part_args, row(bn0_g), row(bn0_b), W1, row(b1),
      row(bn1_g), row(bn1_b), W2, row(b2), row(bn2_g), row(bn2_b), Wo, const)

    return out.reshape(batch)


# speedup vs baseline: 1.3880x; 1.0078x over previous
import functools
import math

import jax
import jax.numpy as jnp
from jax import lax
from jax.experimental import pallas as pl
from jax.experimental.pallas import tpu as pltpu
from jax.experimental.pallas import tpu_sc as plsc

NUM_FIELDS = 26
FIELD_DIM = 100000
DIM = 16
EPS = 1e-5

NC = 2
NS = 16
NW = NC * NS
LANES = 16

TBLK = 131072
SPLITS = [
    (0, NUM_FIELDS, 0, 20),
]


def _transpose_kernel(src_ref, lin_ref, dst_ref, lin_out_ref):
    blk = src_ref.shape[1]
    for k in range(blk // 1024):
        x8 = jnp.concatenate(
            [src_ref[:, k * 1024 + j * 128: k * 1024 + (j + 1) * 128]
             for j in range(8)], axis=0)
        dst_ref[k * 128:(k + 1) * 128, :] = x8.T
    for p in range(blk // 1024):
        piece = jnp.concatenate(
            [lin_ref[:, p * 1024 + s * 128: p * 1024 + (s + 1) * 128]
             for s in range(8)], axis=0)
        lin_out_ref[p * 8:(p + 1) * 8, :] = piece


def _to_row_major(emb_t, lin_t, lo, nblk):
    br = TBLK * DIM // 128
    nrow_t2 = nblk * br
    nrow = nblk * (TBLK // 128)
    flat, lin_flat = pl.pallas_call(
        _transpose_kernel,
        grid=(nblk,),
        in_specs=[pl.BlockSpec((DIM, TBLK), lambda i: (0, i + lo)),
                  pl.BlockSpec((1, TBLK), lambda i: (0, i + lo))],
        out_specs=[pl.BlockSpec((br, 128), lambda i: (i, 0)),
                   pl.BlockSpec((TBLK // 128, 128), lambda i: (i, 0))],
        out_shape=[
            jax.ShapeDtypeStruct((nrow_t2, 128), jnp.float32),
            jax.ShapeDtypeStruct((nrow, 128), jnp.float32),
        ],
    )(emb_t, lin_t)
    return (flat.reshape(nrow_t2 * 128 // DIM, DIM),
            lin_flat.reshape(nrow * 128))


def _sc_gather_part(xi_flat, xip_flat, emb_table, lin_flat, batch, nf):
    per_w = batch // NW
    chunk = 64 if nf >= 16 else 128
    n_chunks = per_w // chunk
    rows = chunk * nf
    idx_rows = rows // 128
    mesh = plsc.VectorSubcoreMesh(core_axis_name="c", subcore_axis_name="s")

    @functools.partial(
        pl.kernel,
        out_type=[
            jax.ShapeDtypeStruct((batch, DIM), jnp.float32),
            jax.ShapeDtypeStruct((batch, DIM), jnp.float32),
            jax.ShapeDtypeStruct((batch * nf,), jnp.float32),
        ],
        mesh=mesh,
        compiler_params=pltpu.CompilerParams(use_tc_tiling_on_sc=False),
        scratch_types=[
            pltpu.VMEM((rows,), jnp.int32),
            pltpu.VMEM((rows,), jnp.int32),
            pltpu.VMEM((rows, DIM), jnp.float32),
            pltpu.VMEM((rows,), jnp.float32),
            pltpu.VMEM((chunk, DIM), jnp.float32),
            pltpu.VMEM((chunk, DIM), jnp.float32),
            pltpu.SemaphoreType.DMA,
            pltpu.SemaphoreType.DMA,
        ],
    )
    def sc_kernel(xi_hbm, xip_hbm, emb_hbm, lin_hbm, s_hbm, sq_hbm,
                  linval_hbm, idx_v, idxp_v, rows_v, linv_v, s_v, sq_v,
                  sem_e, sem_l):
        wid = lax.axis_index("s") * NC + lax.axis_index("c")

        def chunk_body(c, _):
            base_e = wid * per_w + c * chunk
            i0 = base_e * nf

            pltpu.sync_copy(xi_hbm.at[pl.ds(i0, rows)], idx_v)
            pltpu.sync_copy(xip_hbm.at[pl.ds(i0, rows)], idxp_v)

            copies = []
            for j in range(idx_rows):
                copies.append(pltpu.async_copy(
                    emb_hbm.at[idxp_v.at[pl.ds(j * 128, 128)]],
                    rows_v.at[pl.ds(j * 128, 128)], sem_e))
            for j in range(idx_rows):
                copies.append(pltpu.async_copy(
                    lin_hbm.at[idx_v.at[pl.ds(j * 128, 128)]],
                    linv_v.at[pl.ds(j * 128, 128)], sem_l))
            for cp in copies:
                cp.wait()

            zero = jnp.zeros((LANES,), jnp.float32)

            def elem_body(e, _):
                s = zero
                sq = zero
                base = e * nf
                for f in range(nf):
                    v = rows_v[base + f]
                    s = s + v
                    sq = sq + v * v
                s_v[e] = s
                sq_v[e] = sq
                return 0

            lax.fori_loop(0, chunk, elem_body, 0, unroll=False)

            pltpu.sync_copy(s_v, s_hbm.at[pl.ds(base_e, chunk)])
            pltpu.sync_copy(sq_v, sq_hbm.at[pl.ds(base_e, chunk)])
            pltpu.sync_copy(linv_v, linval_hbm.at[pl.ds(i0, rows)])
            return 0

        lax.fori_loop(0, n_chunks, chunk_body, 0, unroll=False)

    return sc_kernel(xi_flat, xip_flat, emb_table, lin_flat)


def _mlp_kernel(nparts, *refs):
    (bn0g_ref, bn0b_ref, w1_ref, b1_ref, bn1g_ref, bn1b_ref, w2_ref,
     b2_ref, bn2g_ref, bn2b_ref, wo_ref, const_ref,
     out_ref) = refs[3 * nparts:]
    inv = jnp.float32(1.0 / math.sqrt(1.0 + EPS))
    s = sum(refs[3 * p][...] for p in range(nparts))
    sq = sum(refs[3 * p + 1][...] for p in range(nparts))
    cross = 0.5 * (s * s - sq)
    lin = sum(jnp.sum(refs[3 * p + 2][...], axis=1, keepdims=True)
              for p in range(nparts))
    xb = cross * (bn0g_ref[...] * inv) + bn0b_ref[...]
    h = jnp.dot(xb, w1_ref[...], preferred_element_type=jnp.float32)
    h = (h + b1_ref[...]) * (bn1g_ref[...] * inv) + bn1b_ref[...]
    h = jnp.maximum(h, 0.0)
    h = jnp.dot(h, w2_ref[...], preferred_element_type=jnp.float32)
    h = (h + b2_ref[...]) * (bn2g_ref[...] * inv) + bn2b_ref[...]
    h = jnp.maximum(h, 0.0)
    o = jnp.dot(h, wo_ref[...], preferred_element_type=jnp.float32)
    out_ref[...] = o + lin + const_ref[...]


def _perm(r):
    return (r // 1024) * 1024 + (r % 128) * 8 + (r // 128) % 8


def kernel(x, emb_table, lin_table, lin_bias, bn0_g, bn0_b, W1, b1,
           bn1_g, bn1_b, W2, b2, bn2_g, bn2_b, Wo, bo):
    batch = x.shape[0]
    offsets = (jnp.arange(NUM_FIELDS) * FIELD_DIM).astype(jnp.int32)
    xi2d = x.astype(jnp.int32) + offsets[None, :]

    parts = []
    for f0, nf, lo, nblk in SPLITS:
        xi_p = xi2d[:, f0:f0 + nf].reshape(-1) - lo * TBLK
        emb_p, lin_p = _to_row_major(emb_table.T, lin_table.T, lo, nblk)
        parts.append(_sc_gather_part(xi_p, _perm(xi_p), emb_p, lin_p,
                                     batch, nf))
    bk = 2048
    grid = (batch // bk,)
    row = lambda a: a.reshape(1, -1)
    blk2 = lambda w: pl.BlockSpec((bk, w), lambda i: (i, 0))
    full = lambda shape: pl.BlockSpec(shape, lambda i: (0, 0))
    const = (lin_bias + bo).reshape(1, 1)

    part_specs = []
    part_args = []
    for (f0, nf, lo, nblk), (s_p, sq_p, lv_p) in zip(SPLITS, parts):
        part_specs += [blk2(DIM), blk2(DIM), blk2(nf)]
        part_args += [s_p, sq_p, lv_p.reshape(batch, nf)]

    out = pl.pallas_call(
        functools.partial(_mlp_kernel, len(SPLITS)),
        grid=grid,
        in_specs=part_specs + [
            full((1, DIM)), full((1, DIM)),
            full((DIM, 64)), full((1, 64)), full((1, 64)), full((1, 64)),
            full((64, 32)), full((1, 32)), full((1, 32)), full((1, 32)),
            full((32, 1)), full((1, 1)),
        ],
        out_specs=pl.BlockSpec((bk, 1), lambda i: (i, 0)),
        out_shape=jax.ShapeDtypeStruct((batch, 1), jnp.float32),
    )(*part_args, row(bn0_g), row(bn0_b), W1, row(b1),
      row(bn1_g), row(bn1_b), W2, row(b2), row(bn2_g), row(bn2_b), Wo, const)

    return out.reshape(batch)
